# Initial kernel scaffold; baseline (speedup 1.0000x reference)
import jax
import jax.numpy as jnp
from jax.experimental import pallas as pl


def kernel(x, edge_index, edge_values):
    raise NotImplementedError("write your pallas kernel here")


# speedup vs baseline: 5.1025x; 5.1025x over previous
import jax
import jax.numpy as jnp
from jax import lax
from jax.experimental import pallas as pl
from jax.experimental.pallas import tpu as pltpu
from jax.experimental.pallas import tpu_sc as plsc

N = 10000
D = 128
H = D // 2
E = 320000
NC = 2
NS = 16
L = 16
K = 128
TE = E // NS
C = -(-TE // K)
P = C * K
NP_ = 10240
RPT = NP_ // NS
ZR = 128


def _sc_body(xs_hbm, src_hbm, dst_hbm, val_hbm, out_hbm,
             gidx_v, dst_v, val_v, rows_v, acc_sh, sem):
    c = lax.axis_index("c")
    s = lax.axis_index("s")

    pltpu.sync_copy(src_hbm.at[s], gidx_v)
    pltpu.sync_copy(dst_hbm.at[s], dst_v)
    pltpu.sync_copy(val_hbm.at[s], val_v)

    def gi_row(r, _):
        for q in range(K // L):
            v = gidx_v[r, pl.ds(q * L, L)]
            gidx_v[r, pl.ds(q * L, L)] = v * 2 + c
        return 0
    lax.fori_loop(0, C, gi_row, 0)

    def z_row(r, _):
        for q in range(H // L):
            rows_v[r, pl.ds(q * L, L)] = jnp.zeros((L,), jnp.float32)
        return 0
    lax.fori_loop(0, ZR, z_row, 0)
    for i in range(RPT // ZR):
        pltpu.sync_copy(rows_v, acc_sh.at[pl.ds(s * RPT + i * ZR, ZR)])

    plsc.subcore_barrier()

    def chunk(j, _):
        pltpu.async_copy(xs_hbm.at[gidx_v.at[j]], rows_v, sem).wait()

        def group(g, _):
            vals16 = val_v[j, pl.ds(g * L, L)]
            for t in range(L):
                vv = jnp.full((L,), vals16[t], jnp.float32)
                e = g * L + t
                for q in range(H // L):
                    rows_v[e, pl.ds(q * L, L)] = (
                        rows_v[e, pl.ds(q * L, L)] * vv)
            return 0
        lax.fori_loop(0, K // L, group, 0)

        pltpu.sync_copy(rows_v, acc_sh.at[dst_v.at[j]], add=True)
        return 0
    lax.fori_loop(0, C, chunk, 0)

    plsc.subcore_barrier()
    r0 = s * RPT
    pltpu.sync_copy(acc_sh.at[pl.ds(r0, RPT)], out_hbm.at[c, pl.ds(r0, RPT)])


_mesh = plsc.VectorSubcoreMesh(core_axis_name="c", subcore_axis_name="s")

_sc_call = pl.kernel(
    _sc_body,
    out_type=jax.ShapeDtypeStruct((NC, NP_, H), jnp.float32),
    mesh=_mesh,
    scratch_types=[
        pltpu.VMEM((C, K), jnp.int32),
        pltpu.VMEM((C, K), jnp.int32),
        pltpu.VMEM((C, K), jnp.float32),
        pltpu.VMEM((K, H), jnp.float32),
        pltpu.VMEM_SHARED((NP_, H), jnp.float32),
        pltpu.SemaphoreType.DMA,
    ],
    compiler_params=pltpu.CompilerParams(use_tc_tiling_on_sc=False),
)


def kernel(x, edge_index, edge_values):
    xs = x.reshape(2 * N, H)
    pad = NS * P - E
    src = jnp.concatenate([edge_index[1], jnp.zeros((pad,), jnp.int32)])
    dst = jnp.concatenate([edge_index[0], jnp.zeros((pad,), jnp.int32)])
    val = jnp.concatenate([edge_values, jnp.zeros((pad,), jnp.float32)])
    out2 = _sc_call(xs,
                    src.reshape(NS, C, K),
                    dst.reshape(NS, C, K),
                    val.reshape(NS, C, K))
    return jnp.concatenate([out2[0, :N], out2[1, :N]], axis=1)


# speedup vs baseline: 7.5444x; 1.4786x over previous
import jax
import jax.numpy as jnp
from jax import lax
from jax.experimental import pallas as pl
from jax.experimental.pallas import tpu as pltpu
from jax.experimental.pallas import tpu_sc as plsc

N = 10000
D = 128
H = D // 2
E = 320000
NC = 2
NS = 16
L = 16
K = 128
TE = E // NS
C = -(-TE // K)
P = C * K
NP_ = 10240
RPT = NP_ // NS
ZR = 128


def _sc_body(xs_hbm, src_hbm, dst_hbm, val_hbm, out_hbm,
             gidx_v, dst_v, val_v, rows_v, acc_sh, sem_g, sem_s):
    c = lax.axis_index("c")
    s = lax.axis_index("s")

    pltpu.sync_copy(src_hbm.at[s], gidx_v)
    pltpu.sync_copy(dst_hbm.at[s], dst_v)
    pltpu.sync_copy(val_hbm.at[s], val_v)

    def gi_row(r, _):
        for q in range(K // L):
            v = gidx_v[r, pl.ds(q * L, L)]
            gidx_v[r, pl.ds(q * L, L)] = v * 2 + c
        return 0
    lax.fori_loop(0, C, gi_row, 0)

    def z_row(r, _):
        for q in range(H // L):
            rows_v[0, r, pl.ds(q * L, L)] = jnp.zeros((L,), jnp.float32)
        return 0
    lax.fori_loop(0, ZR, z_row, 0)
    for i in range(RPT // ZR):
        pltpu.sync_copy(rows_v.at[0],
                        acc_sh.at[pl.ds(s * RPT + i * ZR, ZR)])

    plsc.subcore_barrier()

    pltpu.async_copy(xs_hbm.at[gidx_v.at[0]], rows_v.at[0], sem_g)

    def chunk(j, _):
        b = lax.rem(j, 2)
        nb = 1 - b

        @pl.when(j > 0)
        def _():
            pltpu.make_async_copy(
                rows_v.at[nb], acc_sh.at[dst_v.at[j - 1]], sem_s).wait()

        @pl.when(j + 1 < C)
        def _():
            pltpu.async_copy(xs_hbm.at[gidx_v.at[j + 1]], rows_v.at[nb],
                             sem_g)

        pltpu.make_async_copy(xs_hbm.at[gidx_v.at[j]], rows_v.at[b],
                              sem_g).wait()

        def group(g, _):
            vals16 = val_v[j, pl.ds(g * L, L)]
            for t in range(L):
                vv = jnp.full((L,), vals16[t], jnp.float32)
                e = g * L + t
                for q in range(H // L):
                    rows_v[b, e, pl.ds(q * L, L)] = (
                        rows_v[b, e, pl.ds(q * L, L)] * vv)
            return 0
        lax.fori_loop(0, K // L, group, 0)

        pltpu.async_copy(rows_v.at[b], acc_sh.at[dst_v.at[j]], sem_s,
                         add=True)
        return 0
    lax.fori_loop(0, C, chunk, 0)
    pltpu.make_async_copy(
        rows_v.at[(C - 1) % 2], acc_sh.at[dst_v.at[C - 1]], sem_s).wait()

    plsc.subcore_barrier()
    r0 = s * RPT
    pltpu.sync_copy(acc_sh.at[pl.ds(r0, RPT)], out_hbm.at[c, pl.ds(r0, RPT)])


_mesh = plsc.VectorSubcoreMesh(core_axis_name="c", subcore_axis_name="s")

_sc_call = pl.kernel(
    _sc_body,
    out_type=jax.ShapeDtypeStruct((NC, NP_, H), jnp.float32),
    mesh=_mesh,
    scratch_types=[
        pltpu.VMEM((C, K), jnp.int32),
        pltpu.VMEM((C, K), jnp.int32),
        pltpu.VMEM((C, K), jnp.float32),
        pltpu.VMEM((2, K, H), jnp.float32),
        pltpu.VMEM_SHARED((NP_, H), jnp.float32),
        pltpu.SemaphoreType.DMA,
        pltpu.SemaphoreType.DMA,
    ],
    compiler_params=pltpu.CompilerParams(use_tc_tiling_on_sc=False),
)


def kernel(x, edge_index, edge_values):
    xs = x.reshape(2 * N, H)
    pad = NS * P - E
    src = jnp.concatenate([edge_index[1], jnp.zeros((pad,), jnp.int32)])
    dst = jnp.concatenate([edge_index[0], jnp.zeros((pad,), jnp.int32)])
    val = jnp.concatenate([edge_values, jnp.zeros((pad,), jnp.float32)])
    out2 = _sc_call(xs,
                    src.reshape(NS, C, K),
                    dst.reshape(NS, C, K),
                    val.reshape(NS, C, K))
    return jnp.concatenate([out2[0, :N], out2[1, :N]], axis=1)


# speedup vs baseline: 8.4134x; 1.1152x over previous
import jax
import jax.numpy as jnp
from jax import lax
from jax.experimental import pallas as pl
from jax.experimental.pallas import tpu as pltpu
from jax.experimental.pallas import tpu_sc as plsc

N = 10000
D = 128
H = D // 2
E = 320000
NC = 2
NS = 16
L = 16
K = 128
TE = E // NS
C = -(-TE // K)
P = C * K
NP_ = 10240
RPT = NP_ // NS
ZR = 128


def _sc_body(xs_hbm, src_hbm, dst_hbm, val_hbm, out_hbm,
             gidx_v, dst_v, val_v, rows_v, acc_sh, sem_g, sem_s):
    c = lax.axis_index("c")
    s = lax.axis_index("s")

    pltpu.sync_copy(src_hbm.at[s], gidx_v)
    pltpu.sync_copy(dst_hbm.at[s], dst_v)
    pltpu.sync_copy(val_hbm.at[s], val_v)

    def gi_row(r, _):
        for q in range(K // L):
            v = gidx_v[r, pl.ds(q * L, L)]
            gidx_v[r, pl.ds(q * L, L)] = v * 2 + c
        return 0
    lax.fori_loop(0, C, gi_row, 0)

    def z_row(r, _):
        for q in range(H // L):
            rows_v[0, r, pl.ds(q * L, L)] = jnp.zeros((L,), jnp.float32)
        return 0
    lax.fori_loop(0, ZR, z_row, 0)
    for i in range(RPT // ZR):
        pltpu.sync_copy(rows_v.at[0],
                        acc_sh.at[pl.ds(s * RPT + i * ZR, ZR)])

    plsc.subcore_barrier()

    pltpu.async_copy(xs_hbm.at[gidx_v.at[0]], rows_v.at[0], sem_g)

    def chunk(j, _):
        b = lax.rem(j, 3)
        nb = lax.rem(j + 1, 3)

        @pl.when(j > 1)
        def _():
            pltpu.make_async_copy(
                rows_v.at[nb], acc_sh.at[dst_v.at[j - 2]], sem_s).wait()

        @pl.when(j + 1 < C)
        def _():
            pltpu.async_copy(xs_hbm.at[gidx_v.at[j + 1]], rows_v.at[nb],
                             sem_g)

        pltpu.make_async_copy(xs_hbm.at[gidx_v.at[j]], rows_v.at[b],
                              sem_g).wait()

        def group(g, _):
            vals16 = val_v[j, pl.ds(g * L, L)]
            for t in range(L):
                vv = jnp.full((L,), vals16[t], jnp.float32)
                e = g * L + t
                for q in range(H // L):
                    rows_v[b, e, pl.ds(q * L, L)] = (
                        rows_v[b, e, pl.ds(q * L, L)] * vv)
            return 0
        lax.fori_loop(0, K // L, group, 0)

        pltpu.async_copy(rows_v.at[b], acc_sh.at[dst_v.at[j]], sem_s,
                         add=True)
        return 0
    lax.fori_loop(0, C, chunk, 0)
    pltpu.make_async_copy(
        rows_v.at[(C - 2) % 3], acc_sh.at[dst_v.at[C - 2]], sem_s).wait()
    pltpu.make_async_copy(
        rows_v.at[(C - 1) % 3], acc_sh.at[dst_v.at[C - 1]], sem_s).wait()

    plsc.subcore_barrier()
    r0 = s * RPT
    pltpu.sync_copy(acc_sh.at[pl.ds(r0, RPT)], out_hbm.at[c, pl.ds(r0, RPT)])


_mesh = plsc.VectorSubcoreMesh(core_axis_name="c", subcore_axis_name="s")

_sc_call = pl.kernel(
    _sc_body,
    out_type=jax.ShapeDtypeStruct((NC, NP_, H), jnp.float32),
    mesh=_mesh,
    scratch_types=[
        pltpu.VMEM((C, K), jnp.int32),
        pltpu.VMEM((C, K), jnp.int32),
        pltpu.VMEM((C, K), jnp.float32),
        pltpu.VMEM((3, K, H), jnp.float32),
        pltpu.VMEM_SHARED((NP_, H), jnp.float32),
        pltpu.SemaphoreType.DMA,
        pltpu.SemaphoreType.DMA,
    ],
    compiler_params=pltpu.CompilerParams(use_tc_tiling_on_sc=False),
)


def kernel(x, edge_index, edge_values):
    xs = x.reshape(2 * N, H)
    pad = NS * P - E
    src = jnp.concatenate([edge_index[1], jnp.zeros((pad,), jnp.int32)])
    dst = jnp.concatenate([edge_index[0], jnp.zeros((pad,), jnp.int32)])
    val = jnp.concatenate([edge_values, jnp.zeros((pad,), jnp.float32)])
    out2 = _sc_call(xs,
                    src.reshape(NS, C, K),
                    dst.reshape(NS, C, K),
                    val.reshape(NS, C, K))
    return jnp.concatenate([out2[0, :N], out2[1, :N]], axis=1)


# speedup vs baseline: 12.8618x; 1.5287x over previous
import jax
import jax.numpy as jnp
from jax import lax
from jax.experimental import pallas as pl
from jax.experimental.pallas import tpu as pltpu
from jax.experimental.pallas import tpu_sc as plsc

N = 10000
D = 128
E = 320000
NC = 2
NS = 16
NW = NC * NS
L = 16
K = 80
C = 125
RPT = N // NS
ZR = 125


def _sc_body(x_hbm, src_hbm, dst_hbm, val_hbm, out_hbm,
             src_v, dst_v, val_v, rows_v, acc_sh, sem_g, sem_v, sem_s):
    c = lax.axis_index("c")
    s = lax.axis_index("s")
    w = c * NS + s

    pltpu.sync_copy(src_hbm.at[w], src_v)
    pltpu.sync_copy(dst_hbm.at[w], dst_v)

    def z_row(r, _):
        for q in range(D // L):
            rows_v[r, pl.ds(q * L, L)] = jnp.zeros((L,), jnp.float32)
        return 0
    lax.fori_loop(0, ZR, z_row, 0)
    for i in range(RPT // ZR):
        pltpu.sync_copy(rows_v.at[pl.ds(0, ZR)],
                        acc_sh.at[pl.ds(s * RPT + i * ZR, ZR)])

    plsc.subcore_barrier()

    pltpu.async_copy(val_hbm.at[w, 0], val_v.at[0], sem_v)
    pltpu.async_copy(x_hbm.at[src_v.at[0]], rows_v.at[pl.ds(0, K)], sem_g)

    def chunk(j, _):
        b = lax.rem(j, 3)
        nb = lax.rem(j + 1, 3)

        @pl.when(j > 1)
        def _():
            pltpu.make_async_copy(rows_v.at[pl.ds(nb * K, K)],
                                  acc_sh.at[dst_v.at[j - 2]], sem_s).wait()

        @pl.when(j + 1 < C)
        def _():
            pltpu.async_copy(val_hbm.at[w, j + 1], val_v.at[nb], sem_v)
            pltpu.async_copy(x_hbm.at[src_v.at[j + 1]],
                             rows_v.at[pl.ds(nb * K, K)], sem_g)

        pltpu.make_async_copy(val_hbm.at[w, j], val_v.at[b], sem_v).wait()
        pltpu.make_async_copy(x_hbm.at[src_v.at[j]],
                              rows_v.at[pl.ds(b * K, K)], sem_g).wait()

        def group(g, _):
            vals16 = val_v[b, pl.ds(g * L, L)]
            for t in range(L):
                vv = jnp.full((L,), vals16[t], jnp.float32)
                e = b * K + g * L + t
                got = [rows_v[e, pl.ds(q * L, L)] for q in range(D // L)]
                for q in range(D // L):
                    rows_v[e, pl.ds(q * L, L)] = got[q] * vv
            return 0
        lax.fori_loop(0, K // L, group, 0)

        pltpu.async_copy(rows_v.at[pl.ds(b * K, K)],
                         acc_sh.at[dst_v.at[j]], sem_s, add=True)
        return 0
    lax.fori_loop(0, C, chunk, 0)
    pltpu.make_async_copy(rows_v.at[pl.ds(((C - 2) % 3) * K, K)],
                          acc_sh.at[dst_v.at[C - 2]], sem_s).wait()
    pltpu.make_async_copy(rows_v.at[pl.ds(((C - 1) % 3) * K, K)],
                          acc_sh.at[dst_v.at[C - 1]], sem_s).wait()

    plsc.subcore_barrier()
    r0 = s * RPT
    pltpu.sync_copy(acc_sh.at[pl.ds(r0, RPT)], out_hbm.at[c, pl.ds(r0, RPT)])


_mesh = plsc.VectorSubcoreMesh(core_axis_name="c", subcore_axis_name="s")

_sc_call = pl.kernel(
    _sc_body,
    out_type=jax.ShapeDtypeStruct((NC, N, D), jnp.float32),
    mesh=_mesh,
    scratch_types=[
        pltpu.VMEM((C, K), jnp.int32),
        pltpu.VMEM((C, K), jnp.int32),
        pltpu.VMEM((3, K), jnp.float32),
        pltpu.VMEM((3 * K, D), jnp.float32),
        pltpu.VMEM_SHARED((N, D), jnp.float32),
        pltpu.SemaphoreType.DMA,
        pltpu.SemaphoreType.DMA,
        pltpu.SemaphoreType.DMA,
    ],
    compiler_params=pltpu.CompilerParams(use_tc_tiling_on_sc=False),
)


def _add_body(p_ref, o_ref):
    o_ref[...] = p_ref[0] + p_ref[1]


_tc_add = pl.pallas_call(
    _add_body,
    out_shape=jax.ShapeDtypeStruct((N, D), jnp.float32),
)


def kernel(x, edge_index, edge_values):
    src = edge_index[1].reshape(NW, C, K)
    dst = edge_index[0].reshape(NW, C, K)
    val = edge_values.reshape(NW, C, K)
    partials = _sc_call(x, src, dst, val)
    return _tc_add(partials)


# speedup vs baseline: 13.1290x; 1.0208x over previous
import jax
import jax.numpy as jnp
from jax import lax
from jax.experimental import pallas as pl
from jax.experimental.pallas import tpu as pltpu
from jax.experimental.pallas import tpu_sc as plsc

N = 10000
D = 128
E = 320000
NC = 2
NS = 16
NW = NC * NS
L = 16
K = 80
C = 125
RPT = N // NS
ZR = 125


def _sc_body(x_hbm, src_hbm, dst_hbm, val_hbm, out_hbm,
             src_v, dst_v, val_v, rows_v, acc_sh, sem_g, sem_v, sem_s):
    c = lax.axis_index("c")
    s = lax.axis_index("s")
    w = c * NS + s

    pltpu.async_copy(src_hbm.at[w], src_v, sem_v)
    pltpu.async_copy(dst_hbm.at[w], dst_v, sem_v)

    def z_row(r, _):
        for q in range(D // L):
            rows_v[K + r, pl.ds(q * L, L)] = jnp.zeros((L,), jnp.float32)
        return 0
    lax.fori_loop(0, ZR, z_row, 0)

    pltpu.make_async_copy(src_hbm.at[w], src_v, sem_v).wait()
    pltpu.make_async_copy(dst_hbm.at[w], dst_v, sem_v).wait()

    pltpu.async_copy(val_hbm.at[w, 0], val_v.at[0], sem_v)
    pltpu.async_copy(x_hbm.at[src_v.at[0]], rows_v.at[pl.ds(0, K)], sem_g)

    for i in range(RPT // ZR):
        pltpu.async_copy(rows_v.at[pl.ds(K, ZR)],
                         acc_sh.at[pl.ds(s * RPT + i * ZR, ZR)], sem_s)
    for i in range(RPT // ZR):
        pltpu.make_async_copy(rows_v.at[pl.ds(K, ZR)],
                              acc_sh.at[pl.ds(s * RPT + i * ZR, ZR)],
                              sem_s).wait()

    plsc.subcore_barrier()


    def chunk(j, _):
        b = lax.rem(j, 3)
        nb = lax.rem(j + 1, 3)

        @pl.when(j > 1)
        def _():
            pltpu.make_async_copy(rows_v.at[pl.ds(nb * K, K)],
                                  acc_sh.at[dst_v.at[j - 2]], sem_s).wait()

        @pl.when(j + 1 < C)
        def _():
            pltpu.async_copy(val_hbm.at[w, j + 1], val_v.at[nb], sem_v)
            pltpu.async_copy(x_hbm.at[src_v.at[j + 1]],
                             rows_v.at[pl.ds(nb * K, K)], sem_g)

        pltpu.make_async_copy(val_hbm.at[w, j], val_v.at[b], sem_v).wait()
        pltpu.make_async_copy(x_hbm.at[src_v.at[j]],
                              rows_v.at[pl.ds(b * K, K)], sem_g).wait()

        def group(g, _):
            vals16 = val_v[b, pl.ds(g * L, L)]
            for t in range(L):
                vv = jnp.full((L,), vals16[t], jnp.float32)
                e = b * K + g * L + t
                got = [rows_v[e, pl.ds(q * L, L)] for q in range(D // L)]
                for q in range(D // L):
                    rows_v[e, pl.ds(q * L, L)] = got[q] * vv
            return 0
        lax.fori_loop(0, K // L, group, 0)

        pltpu.async_copy(rows_v.at[pl.ds(b * K, K)],
                         acc_sh.at[dst_v.at[j]], sem_s, add=True)
        return 0
    lax.fori_loop(0, C, chunk, 0)
    pltpu.make_async_copy(rows_v.at[pl.ds(((C - 2) % 3) * K, K)],
                          acc_sh.at[dst_v.at[C - 2]], sem_s).wait()
    pltpu.make_async_copy(rows_v.at[pl.ds(((C - 1) % 3) * K, K)],
                          acc_sh.at[dst_v.at[C - 1]], sem_s).wait()

    plsc.subcore_barrier()
    r0 = s * RPT
    pltpu.sync_copy(acc_sh.at[pl.ds(r0, RPT)], out_hbm.at[c, pl.ds(r0, RPT)])


_mesh = plsc.VectorSubcoreMesh(core_axis_name="c", subcore_axis_name="s")

_sc_call = pl.kernel(
    _sc_body,
    out_type=jax.ShapeDtypeStruct((NC, N, D), jnp.float32),
    mesh=_mesh,
    scratch_types=[
        pltpu.VMEM((C, K), jnp.int32),
        pltpu.VMEM((C, K), jnp.int32),
        pltpu.VMEM((3, K), jnp.float32),
        pltpu.VMEM((3 * K, D), jnp.float32),
        pltpu.VMEM_SHARED((N, D), jnp.float32),
        pltpu.SemaphoreType.DMA,
        pltpu.SemaphoreType.DMA,
        pltpu.SemaphoreType.DMA,
    ],
    compiler_params=pltpu.CompilerParams(use_tc_tiling_on_sc=False),
)


def _add_body(p_ref, o_ref):
    o_ref[...] = p_ref[0] + p_ref[1]


_tc_add = pl.pallas_call(
    _add_body,
    out_shape=jax.ShapeDtypeStruct((N, D), jnp.float32),
)


def kernel(x, edge_index, edge_values):
    src = edge_index[1].reshape(NW, C, K)
    dst = edge_index[0].reshape(NW, C, K)
    val = edge_values.reshape(NW, C, K)
    partials = _sc_call(x, src, dst, val)
    return _tc_add(partials)


# speedup vs baseline: 13.1624x; 1.0025x over previous
import jax
import jax.numpy as jnp
from jax import lax
from jax.experimental import pallas as pl
from jax.experimental.pallas import tpu as pltpu
from jax.experimental.pallas import tpu_sc as plsc

N = 10000
D = 128
E = 320000
NC = 2
NS = 16
NW = NC * NS
L = 16
K = 80
C = 125
RPT = N // NS
ZR = 125


def _sc_body(x_hbm, src_hbm, dst_hbm, val_hbm, out_hbm,
             src_v, dst_v, val_v, rows_v, acc_sh, sem_g, sem_v, sem_s):
    c = lax.axis_index("c")
    s = lax.axis_index("s")
    w = c * NS + s

    pltpu.async_copy(src_hbm.at[w], src_v, sem_v)
    pltpu.async_copy(dst_hbm.at[w], dst_v, sem_v)

    def z_row(r, _):
        for q in range(D // L):
            rows_v[K + r, pl.ds(q * L, L)] = jnp.zeros((L,), jnp.float32)
        return 0
    lax.fori_loop(0, ZR, z_row, 0)

    pltpu.make_async_copy(src_hbm.at[w], src_v, sem_v).wait()
    pltpu.make_async_copy(dst_hbm.at[w], dst_v, sem_v).wait()

    pltpu.async_copy(val_hbm.at[w, 0], val_v.at[0], sem_v)
    pltpu.async_copy(x_hbm.at[src_v.at[0]], rows_v.at[pl.ds(0, K)], sem_g)

    for i in range(RPT // ZR):
        pltpu.async_copy(rows_v.at[pl.ds(K, ZR)],
                         acc_sh.at[pl.ds(s * RPT + i * ZR, ZR)], sem_s)
    for i in range(RPT // ZR):
        pltpu.make_async_copy(rows_v.at[pl.ds(K, ZR)],
                              acc_sh.at[pl.ds(s * RPT + i * ZR, ZR)],
                              sem_s).wait()

    plsc.subcore_barrier()


    def chunk(j, _):
        b = lax.rem(j, 3)
        nb = lax.rem(j + 1, 3)

        @pl.when(j > 1)
        def _():
            pltpu.make_async_copy(rows_v.at[pl.ds(nb * K, K)],
                                  acc_sh.at[dst_v.at[j - 2]], sem_s).wait()

        @pl.when(j + 1 < C)
        def _():
            pltpu.async_copy(val_hbm.at[w, j + 1], val_v.at[nb], sem_v)
            pltpu.async_copy(x_hbm.at[src_v.at[j + 1]],
                             rows_v.at[pl.ds(nb * K, K)], sem_g)

        pltpu.make_async_copy(val_hbm.at[w, j], val_v.at[b], sem_v).wait()
        pltpu.make_async_copy(x_hbm.at[src_v.at[j]],
                              rows_v.at[pl.ds(b * K, K)], sem_g).wait()

        def group(g, _):
            vals16 = val_v[b, pl.ds(g * L, L)]
            for t in range(L):
                vv = jnp.full((L,), vals16[t], jnp.float32)
                e = b * K + g * L + t
                got = [rows_v[e, pl.ds(q * L, L)] for q in range(D // L)]
                for q in range(D // L):
                    rows_v[e, pl.ds(q * L, L)] = got[q] * vv
            return 0
        lax.fori_loop(0, K // L, group, 0)

        pltpu.async_copy(rows_v.at[pl.ds(b * K, K)],
                         acc_sh.at[dst_v.at[j]], sem_s, add=True)
        return 0
    lax.fori_loop(0, C, chunk, 0)
    pltpu.make_async_copy(rows_v.at[pl.ds(((C - 2) % 3) * K, K)],
                          acc_sh.at[dst_v.at[C - 2]], sem_s).wait()
    pltpu.make_async_copy(rows_v.at[pl.ds(((C - 1) % 3) * K, K)],
                          acc_sh.at[dst_v.at[C - 1]], sem_s).wait()

    plsc.subcore_barrier()
    r0 = s * RPT
    pltpu.sync_copy(acc_sh.at[pl.ds(r0, RPT)], out_hbm.at[c, pl.ds(r0, RPT)])


_mesh = plsc.VectorSubcoreMesh(core_axis_name="c", subcore_axis_name="s")

_sc_call = pl.kernel(
    _sc_body,
    out_type=jax.ShapeDtypeStruct((NC, N, D), jnp.float32),
    mesh=_mesh,
    scratch_types=[
        pltpu.VMEM((C, K), jnp.int32),
        pltpu.VMEM((C, K), jnp.int32),
        pltpu.VMEM((3, K), jnp.float32),
        pltpu.VMEM((3 * K, D), jnp.float32),
        pltpu.VMEM_SHARED((N, D), jnp.float32),
        pltpu.SemaphoreType.DMA,
        pltpu.SemaphoreType.DMA,
        pltpu.SemaphoreType.DMA,
    ],
    compiler_params=pltpu.CompilerParams(
        use_tc_tiling_on_sc=False,
        disable_bounds_checks=True,
        disable_semaphore_checks=True,
    ),
)


def _add_body(p_ref, o_ref):
    o_ref[...] = p_ref[0] + p_ref[1]


_tc_add = pl.pallas_call(
    _add_body,
    out_shape=jax.ShapeDtypeStruct((N, D), jnp.float32),
)


def kernel(x, edge_index, edge_values):
    src = edge_index[1].reshape(NW, C, K)
    dst = edge_index[0].reshape(NW, C, K)
    val = edge_values.reshape(NW, C, K)
    partials = _sc_call(x, src, dst, val)
    return _tc_add(partials)
